# baseline (speedup 1.0000x reference)
import functools
import math

import jax
import jax.numpy as jnp
from jax import lax
from jax.experimental import pallas as pl
from jax.experimental.pallas import tpu as pltpu
from jax.experimental.pallas import tpu_sc as plsc

D_MODEL = 64
SCALE = math.sqrt(D_MODEL)

NC = 2
NS = 16
NW = NC * NS
LANES = 16

CHUNK = 512


@functools.partial(jax.jit, static_argnames=("b_total",))
def _embed(idx_flat, table, *, b_total):
    b_per_w = b_total // NW
    n_chunks = b_per_w // CHUNK
    mesh = plsc.VectorSubcoreMesh(core_axis_name="c", subcore_axis_name="s")

    @functools.partial(
        pl.kernel,
        mesh=mesh,
        out_type=jax.ShapeDtypeStruct((b_total, D_MODEL), jnp.float32),
        scratch_types=[
            pltpu.VMEM((CHUNK,), jnp.int32),
            pltpu.VMEM((CHUNK, D_MODEL), jnp.float32),
            pltpu.SemaphoreType.DMA,
        ],
        compiler_params=pltpu.CompilerParams(use_tc_tiling_on_sc=False),
    )
    def k(idx_hbm, table_hbm, out_hbm, idx_v, rows_v, sem):
        wid = lax.axis_index("s") * NC + lax.axis_index("c")
        base = wid * b_per_w

        @pl.loop(0, n_chunks)
        def _chunk(c):
            row0 = base + c * CHUNK
            pltpu.sync_copy(idx_hbm.at[pl.ds(row0, CHUNK)], idx_v)
            pltpu.async_copy(table_hbm.at[idx_v], rows_v, sem).wait()

            @pl.loop(0, CHUNK)
            def _scale(r):
                for j in range(D_MODEL // LANES):
                    sl = pl.ds(j * LANES, LANES)
                    rows_v[r, sl] = rows_v[r, sl] * SCALE

            pltpu.sync_copy(rows_v, out_hbm.at[pl.ds(row0, CHUNK)])

    return k(idx_flat, table)


def kernel(input_ids, table):
    shape = input_ids.shape
    idx_flat = input_ids.reshape(-1).astype(jnp.int32)
    out = _embed(idx_flat, table, b_total=idx_flat.shape[0])
    return out.reshape(*shape, D_MODEL)


# speedup vs baseline: 1.1405x; 1.1405x over previous
import functools
import math

import jax
import jax.numpy as jnp
from jax import lax
from jax.experimental import pallas as pl
from jax.experimental.pallas import tpu as pltpu
from jax.experimental.pallas import tpu_sc as plsc

D_MODEL = 64
SCALE = math.sqrt(D_MODEL)

NC = 2
NS = 16
NW = NC * NS
LANES = 16
NBUF = 4


@functools.partial(jax.jit, static_argnames=("n_seq", "seq_len"))
def _embed(idx_flat, table, *, n_seq, seq_len):
    seq_per_w = n_seq // NW
    mesh = plsc.VectorSubcoreMesh(core_axis_name="c", subcore_axis_name="s")

    @functools.partial(
        pl.kernel,
        mesh=mesh,
        out_type=jax.ShapeDtypeStruct((n_seq, seq_len, D_MODEL), jnp.float32),
        scratch_types=[
            pltpu.VMEM((seq_per_w * seq_len,), jnp.int32),
            pltpu.VMEM((NBUF, seq_len, D_MODEL), jnp.float32),
            pltpu.SemaphoreType.DMA,
            *([pltpu.SemaphoreType.DMA] * NBUF),
            *([pltpu.SemaphoreType.DMA] * NBUF),
        ],
        compiler_params=pltpu.CompilerParams(use_tc_tiling_on_sc=False),
    )
    def k(idx_hbm, table_hbm, out_hbm, idx_v, rows_v, isem, gs0, gs1, gs2,
          gs3, ss0, ss1, ss2, ss3):
        gsem = (gs0, gs1, gs2, gs3)
        ssem = (ss0, ss1, ss2, ss3)
        wid = lax.axis_index("s") * NC + lax.axis_index("c")
        seq0 = wid * seq_per_w

        pltpu.async_copy(
            idx_hbm.at[pl.ds(seq0 * seq_len, seq_per_w * seq_len)], idx_v,
            isem).wait()

        def idx_slice(c):
            return idx_v.at[pl.ds(c * seq_len, seq_len)]

        def gather(c, b):
            return pltpu.make_async_copy(
                table_hbm.at[idx_slice(c)], rows_v.at[b], gsem[b])

        def scatter(c, b):
            return pltpu.make_async_copy(
                rows_v.at[b], out_hbm.at[seq0 + c], ssem[b])

        gather(0, 0).start()
        gather(1, 1).start()

        @pl.loop(0, seq_per_w, step=NBUF)
        def _outer(t):
            for b in range(NBUF):
                c = t + b
                f = (b + 2) % NBUF
                cn = c + 2

                @pl.when(cn < seq_per_w)
                def _prefetch():
                    @pl.when(cn >= NBUF)
                    def _drain():
                        scatter(cn - NBUF, f).wait()

                    gather(cn, f).start()

                gather(c, b).wait()
                buf = rows_v.at[b]

                @pl.loop(0, seq_len)
                def _scale(r):
                    for j in range(D_MODEL // LANES):
                        sl = pl.ds(j * LANES, LANES)
                        buf[r, sl] = buf[r, sl] * SCALE

                scatter(c, b).start()

        for b in range(NBUF):
            scatter(seq_per_w - NBUF + b, b).wait()

    return k(idx_flat, table)


def kernel(input_ids, table):
    n_seq, seq_len = input_ids.shape
    idx_flat = input_ids.reshape(-1).astype(jnp.int32)
    return _embed(idx_flat, table, n_seq=n_seq, seq_len=seq_len)
